# Initial kernel scaffold; baseline (speedup 1.0000x reference)
import jax
import jax.numpy as jnp
from jax.experimental import pallas as pl


def kernel(x, in_proj_w, in_proj_b, out_proj_w, out_proj_b, ln1_g, ln1_b, ln2_g, ln2_b, router_w, wi, wo):
    raise NotImplementedError("write your pallas kernel here")


# speedup vs baseline: 3.1281x; 3.1281x over previous
import functools

import jax
import jax.numpy as jnp
from jax.experimental import pallas as pl
from jax.experimental.pallas import tpu as pltpu

_H = 16
_CAP = 320
_EPS = 1e-5
_INTERPRET = False


def _dot(a, b, dims):
    return jax.lax.dot_general(a, b, (dims, ((), ())),
                               preferred_element_type=jnp.float32)


def _qkv_kernel(x_ref, w_ref, b_ref, o_ref):
    o_ref[...] = _dot(x_ref[...], w_ref[...], ((1,), (1,))) + b_ref[...]


def _attn_kernel(q_ref, k_ref, v_ref, o_ref, *, scale):
    s = _dot(q_ref[0], k_ref[0], ((1,), (1,))) * scale
    m = jnp.max(s, axis=-1, keepdims=True)
    p = jnp.exp(s - m)
    a = p / jnp.sum(p, axis=-1, keepdims=True)
    o_ref[0] = _dot(a, v_ref[0], ((1,), (0,)))


def _post_kernel(o2_ref, x_ref, wo_ref, bo_ref, g_ref, b_ref, wr_ref,
                 x1_ref, lg_ref, *, eps):
    attn = _dot(o2_ref[...], wo_ref[...], ((1,), (1,))) + bo_ref[...]
    t = x_ref[...] + attn
    m = jnp.mean(t, axis=-1, keepdims=True)
    v = jnp.mean((t - m) * (t - m), axis=-1, keepdims=True)
    x1 = (t - m) / jnp.sqrt(v + eps) * g_ref[...] + b_ref[...]
    x1_ref[...] = x1
    lg_ref[...] = _dot(x1, wr_ref[...], ((1,), (1,)))


def _route_kernel(lg_ref, eidx_ref, maxp_ref, kslot_ref, dest_ref,
                  routed_ref, oh_ref, *, cap, n_exp, s_len, chunk):
    l = lg_ref[...]
    m = jnp.max(l, axis=-1, keepdims=True)
    iota_e = jax.lax.broadcasted_iota(jnp.int32, l.shape, 1)
    cand = jnp.where(l == m, iota_e, n_exp)
    eidx = jnp.min(cand, axis=-1, keepdims=True)
    maxp = 1.0 / jnp.sum(jnp.exp(l - m), axis=-1, keepdims=True)
    onehot = (iota_e == eidx).astype(jnp.float32)
    eidx_ref[...] = eidx
    maxp_ref[...] = maxp
    oh_ref[...] = onehot

    ri = jax.lax.broadcasted_iota(jnp.int32, (chunk, chunk), 0)
    ci = jax.lax.broadcasted_iota(jnp.int32, (chunk, chunk), 1)
    tri = (ci <= ri).astype(jnp.float32)

    def body(c, base):
        sl = pl.ds(c * chunk, chunk)
        oh = oh_ref[sl, :]
        prio = _dot(tri, oh, ((1,), (0,))) + base
        myp = jnp.sum(prio * oh, axis=-1, keepdims=True)
        kslot = myp.astype(jnp.int32) - 1
        eix = eidx_ref[sl, :]
        kslot_ref[sl, :] = kslot
        dest_ref[sl, :] = eix * cap + kslot
        routed_ref[sl, :] = (kslot < cap).astype(jnp.int32)
        return base + jnp.sum(oh, axis=0, keepdims=True)

    jax.lax.fori_loop(0, s_len // chunk, body,
                      jnp.zeros((1, n_exp), jnp.float32))


def _gelu(x):
    return 0.5 * x * (1.0 + jax.lax.erf(x * 0.7071067811865476))


def _ffn_kernel(eidx_ref, kslot_ref, x1_ref, wi_ref, wo_ref, h_ref, xe_ref,
                *, cap, s_len):
    e = pl.program_id(0)
    f = pl.program_id(1)

    @pl.when(f == 0)
    def _():
        sel = (eidx_ref[...] == e) & (kslot_ref[...] < cap)
        iota_k = jax.lax.broadcasted_iota(jnp.int32, (cap, s_len), 0)
        disp = ((kslot_ref[...] == iota_k) & sel).astype(jnp.float32)
        xe_ref[...] = _dot(disp, x1_ref[...], ((1,), (0,)))

    hmid = _gelu(_dot(xe_ref[...], wi_ref[0], ((1,), (1,))))
    part = _dot(hmid, wo_ref[0], ((1,), (1,)))

    @pl.when(f == 0)
    def _():
        h_ref[0] = part

    @pl.when(f > 0)
    def _():
        h_ref[0] = h_ref[0] + part


def _combine_kernel(x1_ref, maxp_ref, dest_ref, routed_ref, h_ref,
                    g_ref, b_ref, o_ref, *, nslot, eps):
    sb = x1_ref.shape[0]
    iota_s = jax.lax.broadcasted_iota(jnp.int32, (sb, nslot), 1)
    routed = routed_ref[...] > 0
    q = ((iota_s == dest_ref[...]) & routed).astype(jnp.float32)
    scat = _dot(q, h_ref[...], ((1,), (0,)))
    nxt = scat + jnp.where(routed, 0.0, 1.0) * x1_ref[...]
    t = x1_ref[...] + maxp_ref[...] * nxt
    m = jnp.mean(t, axis=-1, keepdims=True)
    v = jnp.mean((t - m) * (t - m), axis=-1, keepdims=True)
    o_ref[...] = (t - m) / jnp.sqrt(v + eps) * g_ref[...] + b_ref[...]


def kernel(x, in_proj_w, in_proj_b, out_proj_w, out_proj_b,
           ln1_g, ln1_b, ln2_g, ln2_b, router_w, wi, wo):
    B, S, D = x.shape
    E = router_w.shape[0]
    FF = wi.shape[1]
    H = _H
    DH = D // H
    CAP = _CAP
    NSLOT = E * CAP
    SB = min(256, S)
    NB = S // SB
    NFF = 2
    FFB = FF // NFF

    x2 = x.reshape(S, D)

    qkv = pl.pallas_call(
        _qkv_kernel,
        grid=(NB,),
        in_specs=[
            pl.BlockSpec((SB, D), lambda i: (i, 0)),
            pl.BlockSpec((3 * D, D), lambda i: (0, 0)),
            pl.BlockSpec((1, 3 * D), lambda i: (0, 0)),
        ],
        out_specs=pl.BlockSpec((SB, 3 * D), lambda i: (i, 0)),
        out_shape=jax.ShapeDtypeStruct((S, 3 * D), jnp.float32),
        interpret=_INTERPRET,
    )(x2, in_proj_w, in_proj_b.reshape(1, 3 * D))

    q = qkv[:, :D].reshape(S, H, DH).transpose(1, 0, 2)
    k = qkv[:, D:2 * D].reshape(S, H, DH).transpose(1, 0, 2)
    v = qkv[:, 2 * D:].reshape(S, H, DH).transpose(1, 0, 2)

    o = pl.pallas_call(
        functools.partial(_attn_kernel, scale=1.0 / (DH ** 0.5)),
        grid=(H,),
        in_specs=[pl.BlockSpec((1, S, DH), lambda h: (h, 0, 0))] * 3,
        out_specs=pl.BlockSpec((1, S, DH), lambda h: (h, 0, 0)),
        out_shape=jax.ShapeDtypeStruct((H, S, DH), jnp.float32),
        interpret=_INTERPRET,
    )(q, k, v)
    o2 = o.transpose(1, 0, 2).reshape(S, D)

    x1, logits = pl.pallas_call(
        functools.partial(_post_kernel, eps=_EPS),
        grid=(NB,),
        in_specs=[
            pl.BlockSpec((SB, D), lambda i: (i, 0)),
            pl.BlockSpec((SB, D), lambda i: (i, 0)),
            pl.BlockSpec((D, D), lambda i: (0, 0)),
            pl.BlockSpec((1, D), lambda i: (0, 0)),
            pl.BlockSpec((1, D), lambda i: (0, 0)),
            pl.BlockSpec((1, D), lambda i: (0, 0)),
            pl.BlockSpec((E, D), lambda i: (0, 0)),
        ],
        out_specs=[
            pl.BlockSpec((SB, D), lambda i: (i, 0)),
            pl.BlockSpec((SB, E), lambda i: (i, 0)),
        ],
        out_shape=[
            jax.ShapeDtypeStruct((S, D), jnp.float32),
            jax.ShapeDtypeStruct((S, E), jnp.float32),
        ],
        interpret=_INTERPRET,
    )(o2, x2, out_proj_w, out_proj_b.reshape(1, D), ln1_g.reshape(1, D),
      ln1_b.reshape(1, D), router_w)

    eidx_c, maxp_c, kslot_c, dest_c, routed_c = pl.pallas_call(
        functools.partial(_route_kernel, cap=CAP, n_exp=E, s_len=S,
                          chunk=SB),
        out_shape=[
            jax.ShapeDtypeStruct((S, 1), jnp.int32),
            jax.ShapeDtypeStruct((S, 1), jnp.float32),
            jax.ShapeDtypeStruct((S, 1), jnp.int32),
            jax.ShapeDtypeStruct((S, 1), jnp.int32),
            jax.ShapeDtypeStruct((S, 1), jnp.int32),
        ],
        scratch_shapes=[pltpu.VMEM((S, E), jnp.float32)],
        interpret=_INTERPRET,
    )(logits)

    eidx_row = eidx_c.reshape(1, S)
    kslot_row = kslot_c.reshape(1, S)

    h = pl.pallas_call(
        functools.partial(_ffn_kernel, cap=CAP, s_len=S),
        grid=(E, NFF),
        in_specs=[
            pl.BlockSpec((1, S), lambda e, f: (0, 0)),
            pl.BlockSpec((1, S), lambda e, f: (0, 0)),
            pl.BlockSpec((S, D), lambda e, f: (0, 0)),
            pl.BlockSpec((1, FFB, D), lambda e, f: (e, f, 0)),
            pl.BlockSpec((1, D, FFB), lambda e, f: (e, 0, f)),
        ],
        out_specs=pl.BlockSpec((1, CAP, D), lambda e, f: (e, 0, 0)),
        out_shape=jax.ShapeDtypeStruct((E, CAP, D), jnp.float32),
        scratch_shapes=[pltpu.VMEM((CAP, D), jnp.float32)],
        interpret=_INTERPRET,
    )(eidx_row, kslot_row, x1, wi, wo)
    h_flat = h.reshape(NSLOT, D)

    out = pl.pallas_call(
        functools.partial(_combine_kernel, nslot=NSLOT, eps=_EPS),
        grid=(NB,),
        in_specs=[
            pl.BlockSpec((SB, D), lambda i: (i, 0)),
            pl.BlockSpec((SB, 1), lambda i: (i, 0)),
            pl.BlockSpec((SB, 1), lambda i: (i, 0)),
            pl.BlockSpec((SB, 1), lambda i: (i, 0)),
            pl.BlockSpec((NSLOT, D), lambda i: (0, 0)),
            pl.BlockSpec((1, D), lambda i: (0, 0)),
            pl.BlockSpec((1, D), lambda i: (0, 0)),
        ],
        out_specs=pl.BlockSpec((SB, D), lambda i: (i, 0)),
        out_shape=jax.ShapeDtypeStruct((S, D), jnp.float32),
        interpret=_INTERPRET,
    )(x1, maxp_c, dest_c, routed_c, h_flat, ln2_g.reshape(1, D),
      ln2_b.reshape(1, D))

    return (out.reshape(B, S, D), logits.reshape(B, S, E),
            eidx_c.reshape(B, S))


# speedup vs baseline: 4.3007x; 1.3748x over previous
import functools

import jax
import jax.numpy as jnp
from jax.experimental import pallas as pl
from jax.experimental.pallas import tpu as pltpu

_H = 16
_CAP = 320
_EPS = 1e-5
_INTERPRET = False


def _dot(a, b, dims):
    return jax.lax.dot_general(a, b, (dims, ((), ())),
                               preferred_element_type=jnp.float32)


def _qkv_kernel(x_ref, w_ref, b_ref, o_ref):
    o_ref[...] = _dot(x_ref[...], w_ref[...], ((1,), (1,))) + b_ref[...]


def _attn_kernel(q_ref, k_ref, v_ref, o_ref, *, scale, dh):
    nh = q_ref.shape[1] // dh
    for i in range(nh):
        sl = slice(i * dh, (i + 1) * dh)
        s = _dot(q_ref[:, sl], k_ref[:, sl], ((1,), (1,))) * scale
        m = jnp.max(s, axis=-1, keepdims=True)
        p = jnp.exp(s - m)
        a = p / jnp.sum(p, axis=-1, keepdims=True)
        o_ref[:, sl] = _dot(a, v_ref[:, sl], ((1,), (0,)))


def _post_kernel(o2_ref, x_ref, wo_ref, bo_ref, g_ref, b_ref, wr_ref,
                 x1_ref, lg_ref, *, eps):
    attn = _dot(o2_ref[...], wo_ref[...], ((1,), (1,))) + bo_ref[...]
    t = x_ref[...] + attn
    m = jnp.mean(t, axis=-1, keepdims=True)
    v = jnp.mean((t - m) * (t - m), axis=-1, keepdims=True)
    x1 = (t - m) / jnp.sqrt(v + eps) * g_ref[...] + b_ref[...]
    x1_ref[...] = x1
    lg_ref[...] = _dot(x1, wr_ref[...], ((1,), (1,)))


def _route_kernel(lg_ref, eidx_ref, maxp_ref, kslot_ref, dest_ref,
                  routed_ref, oh_ref, *, cap, n_exp, s_len, chunk):
    l = lg_ref[...]
    m = jnp.max(l, axis=-1, keepdims=True)
    iota_e = jax.lax.broadcasted_iota(jnp.int32, l.shape, 1)
    cand = jnp.where(l == m, iota_e, n_exp)
    eidx = jnp.min(cand, axis=-1, keepdims=True)
    maxp = 1.0 / jnp.sum(jnp.exp(l - m), axis=-1, keepdims=True)
    onehot = (iota_e == eidx).astype(jnp.float32)
    eidx_ref[...] = eidx
    maxp_ref[...] = maxp
    oh_ref[...] = onehot

    ri = jax.lax.broadcasted_iota(jnp.int32, (chunk, chunk), 0)
    ci = jax.lax.broadcasted_iota(jnp.int32, (chunk, chunk), 1)
    tri = (ci <= ri).astype(jnp.float32)

    def body(c, base):
        sl = pl.ds(c * chunk, chunk)
        oh = oh_ref[sl, :]
        prio = _dot(tri, oh, ((1,), (0,))) + base
        myp = jnp.sum(prio * oh, axis=-1, keepdims=True)
        kslot = myp.astype(jnp.int32) - 1
        eix = eidx_ref[sl, :]
        kslot_ref[sl, :] = kslot
        dest_ref[sl, :] = eix * cap + kslot
        routed_ref[sl, :] = (kslot < cap).astype(jnp.int32)
        return base + jnp.sum(oh, axis=0, keepdims=True)

    jax.lax.fori_loop(0, s_len // chunk, body,
                      jnp.zeros((1, n_exp), jnp.float32))


def _gelu(x):
    return 0.5 * x * (1.0 + jax.lax.erf(x * 0.7071067811865476))


def _ffn_kernel(eidx_ref, kslot_ref, x1_ref, wi_ref, wo_ref, h_ref, xe_ref,
                *, cap, s_len):
    e = pl.program_id(0)
    f = pl.program_id(1)

    @pl.when(f == 0)
    def _():
        sel = (eidx_ref[...] == e) & (kslot_ref[...] < cap)
        iota_k = jax.lax.broadcasted_iota(jnp.int32, (cap, s_len), 0)
        disp = ((kslot_ref[...] == iota_k) & sel).astype(jnp.float32)
        xe_ref[...] = _dot(disp, x1_ref[...], ((1,), (0,)))

    hmid = _gelu(_dot(xe_ref[...], wi_ref[0], ((1,), (1,))))
    part = _dot(hmid, wo_ref[0], ((1,), (1,)))

    @pl.when(f == 0)
    def _():
        h_ref[0] = part

    @pl.when(f > 0)
    def _():
        h_ref[0] = h_ref[0] + part


def _combine_kernel(x1_ref, maxp_ref, dest_ref, routed_ref, h_ref,
                    g_ref, b_ref, o_ref, *, nslot, eps):
    sb = x1_ref.shape[0]
    iota_s = jax.lax.broadcasted_iota(jnp.int32, (sb, nslot), 1)
    routed = routed_ref[...] > 0
    q = ((iota_s == dest_ref[...]) & routed).astype(jnp.float32)
    scat = _dot(q, h_ref[...], ((1,), (0,)))
    nxt = scat + jnp.where(routed, 0.0, 1.0) * x1_ref[...]
    t = x1_ref[...] + maxp_ref[...] * nxt
    m = jnp.mean(t, axis=-1, keepdims=True)
    v = jnp.mean((t - m) * (t - m), axis=-1, keepdims=True)
    o_ref[...] = (t - m) / jnp.sqrt(v + eps) * g_ref[...] + b_ref[...]


def kernel(x, in_proj_w, in_proj_b, out_proj_w, out_proj_b,
           ln1_g, ln1_b, ln2_g, ln2_b, router_w, wi, wo):
    B, S, D = x.shape
    E = router_w.shape[0]
    FF = wi.shape[1]
    H = _H
    DH = D // H
    CAP = _CAP
    NSLOT = E * CAP
    SB = min(256, S)
    NB = S // SB
    NFF = 2
    FFB = FF // NFF

    x2 = x.reshape(S, D)

    qkv = pl.pallas_call(
        _qkv_kernel,
        grid=(NB,),
        in_specs=[
            pl.BlockSpec((SB, D), lambda i: (i, 0)),
            pl.BlockSpec((3 * D, D), lambda i: (0, 0)),
            pl.BlockSpec((1, 3 * D), lambda i: (0, 0)),
        ],
        out_specs=pl.BlockSpec((SB, 3 * D), lambda i: (i, 0)),
        out_shape=jax.ShapeDtypeStruct((S, 3 * D), jnp.float32),
        interpret=_INTERPRET,
    )(x2, in_proj_w, in_proj_b.reshape(1, 3 * D))

    HB = 128
    NHB = D // HB
    o2 = pl.pallas_call(
        functools.partial(_attn_kernel, scale=1.0 / (DH ** 0.5), dh=DH),
        grid=(NHB,),
        in_specs=[
            pl.BlockSpec((S, HB), lambda p: (0, p)),
            pl.BlockSpec((S, HB), lambda p: (0, NHB + p)),
            pl.BlockSpec((S, HB), lambda p: (0, 2 * NHB + p)),
        ],
        out_specs=pl.BlockSpec((S, HB), lambda p: (0, p)),
        out_shape=jax.ShapeDtypeStruct((S, D), jnp.float32),
        interpret=_INTERPRET,
    )(qkv, qkv, qkv)

    x1, logits = pl.pallas_call(
        functools.partial(_post_kernel, eps=_EPS),
        grid=(NB,),
        in_specs=[
            pl.BlockSpec((SB, D), lambda i: (i, 0)),
            pl.BlockSpec((SB, D), lambda i: (i, 0)),
            pl.BlockSpec((D, D), lambda i: (0, 0)),
            pl.BlockSpec((1, D), lambda i: (0, 0)),
            pl.BlockSpec((1, D), lambda i: (0, 0)),
            pl.BlockSpec((1, D), lambda i: (0, 0)),
            pl.BlockSpec((E, D), lambda i: (0, 0)),
        ],
        out_specs=[
            pl.BlockSpec((SB, D), lambda i: (i, 0)),
            pl.BlockSpec((SB, E), lambda i: (i, 0)),
        ],
        out_shape=[
            jax.ShapeDtypeStruct((S, D), jnp.float32),
            jax.ShapeDtypeStruct((S, E), jnp.float32),
        ],
        interpret=_INTERPRET,
    )(o2, x2, out_proj_w, out_proj_b.reshape(1, D), ln1_g.reshape(1, D),
      ln1_b.reshape(1, D), router_w)

    eidx_c, maxp_c, kslot_c, dest_c, routed_c = pl.pallas_call(
        functools.partial(_route_kernel, cap=CAP, n_exp=E, s_len=S,
                          chunk=SB),
        out_shape=[
            jax.ShapeDtypeStruct((S, 1), jnp.int32),
            jax.ShapeDtypeStruct((S, 1), jnp.float32),
            jax.ShapeDtypeStruct((S, 1), jnp.int32),
            jax.ShapeDtypeStruct((S, 1), jnp.int32),
            jax.ShapeDtypeStruct((S, 1), jnp.int32),
        ],
        scratch_shapes=[pltpu.VMEM((S, E), jnp.float32)],
        interpret=_INTERPRET,
    )(logits)

    eidx_row = eidx_c.reshape(1, S)
    kslot_row = kslot_c.reshape(1, S)

    h = pl.pallas_call(
        functools.partial(_ffn_kernel, cap=CAP, s_len=S),
        grid=(E, NFF),
        in_specs=[
            pl.BlockSpec((1, S), lambda e, f: (0, 0)),
            pl.BlockSpec((1, S), lambda e, f: (0, 0)),
            pl.BlockSpec((S, D), lambda e, f: (0, 0)),
            pl.BlockSpec((1, FFB, D), lambda e, f: (e, f, 0)),
            pl.BlockSpec((1, D, FFB), lambda e, f: (e, 0, f)),
        ],
        out_specs=pl.BlockSpec((1, CAP, D), lambda e, f: (e, 0, 0)),
        out_shape=jax.ShapeDtypeStruct((E, CAP, D), jnp.float32),
        scratch_shapes=[pltpu.VMEM((CAP, D), jnp.float32)],
        interpret=_INTERPRET,
    )(eidx_row, kslot_row, x1, wi, wo)
    h_flat = h.reshape(NSLOT, D)

    out = pl.pallas_call(
        functools.partial(_combine_kernel, nslot=NSLOT, eps=_EPS),
        grid=(NB,),
        in_specs=[
            pl.BlockSpec((SB, D), lambda i: (i, 0)),
            pl.BlockSpec((SB, 1), lambda i: (i, 0)),
            pl.BlockSpec((SB, 1), lambda i: (i, 0)),
            pl.BlockSpec((SB, 1), lambda i: (i, 0)),
            pl.BlockSpec((NSLOT, D), lambda i: (0, 0)),
            pl.BlockSpec((1, D), lambda i: (0, 0)),
            pl.BlockSpec((1, D), lambda i: (0, 0)),
        ],
        out_specs=pl.BlockSpec((SB, D), lambda i: (i, 0)),
        out_shape=jax.ShapeDtypeStruct((S, D), jnp.float32),
        interpret=_INTERPRET,
    )(x1, maxp_c, dest_c, routed_c, h_flat, ln2_g.reshape(1, D),
      ln2_b.reshape(1, D))

    return (out.reshape(B, S, D), logits.reshape(B, S, E),
            eidx_c.reshape(B, S))


# speedup vs baseline: 4.4014x; 1.0234x over previous
import functools

import jax
import jax.numpy as jnp
from jax import lax
from jax.experimental import pallas as pl
from jax.experimental.pallas import tpu as pltpu
from jax.experimental.pallas import tpu_sc as plsc

_H = 16
_CAP = 320
_EPS = 1e-5
_INTERPRET = False


def _dot(a, b, dims, prec=None):
    return jax.lax.dot_general(a, b, (dims, ((), ())),
                               preferred_element_type=jnp.float32,
                               precision=prec)


def _qkv_kernel(x_ref, w_ref, b_ref, o_ref):
    o_ref[...] = _dot(x_ref[...], w_ref[...], ((1,), (1,))) + b_ref[...]


def _attn_kernel(q_ref, k_ref, v_ref, o_ref, *, scale, dh):
    nh = q_ref.shape[1] // dh
    for i in range(nh):
        sl = slice(i * dh, (i + 1) * dh)
        s = _dot(q_ref[:, sl], k_ref[:, sl], ((1,), (1,))) * scale
        m = jnp.max(s, axis=-1, keepdims=True)
        p = jnp.exp(s - m)
        c = jnp.sum(p, axis=-1, keepdims=True)
        o_ref[:, sl] = _dot(p, v_ref[:, sl], ((1,), (0,))) / c


def _post_kernel(o2_ref, x_ref, wo_ref, bo_ref, g_ref, b_ref, wr_ref,
                 x1_ref, lg_ref, *, eps):
    attn = _dot(o2_ref[...], wo_ref[...], ((1,), (1,))) + bo_ref[...]
    t = x_ref[...] + attn
    m = jnp.mean(t, axis=-1, keepdims=True)
    v = jnp.mean((t - m) * (t - m), axis=-1, keepdims=True)
    x1 = (t - m) / jnp.sqrt(v + eps) * g_ref[...] + b_ref[...]
    x1_ref[...] = x1
    lg_ref[...] = _dot(x1, wr_ref[...], ((1,), (1,)))


def _route_kernel(lg_ref, eidx_ref, maxp_ref, destc_ref, gdest_ref,
                  routed_ref, oh_ref, *, cap, n_exp, s_len, chunk,
                  nslot, sc_chunk):
    l = lg_ref[...]
    m = jnp.max(l, axis=-1, keepdims=True)
    iota_e = jax.lax.broadcasted_iota(jnp.int32, l.shape, 1)
    cand = jnp.where(l == m, iota_e, n_exp)
    eidx = jnp.min(cand, axis=-1, keepdims=True)
    maxp = 1.0 / jnp.sum(jnp.exp(l - m), axis=-1, keepdims=True)
    onehot = (iota_e == eidx).astype(jnp.float32)
    eidx_ref[...] = eidx
    maxp_ref[...] = maxp
    oh_ref[...] = onehot

    ri = jax.lax.broadcasted_iota(jnp.int32, (chunk, chunk), 0)
    ci = jax.lax.broadcasted_iota(jnp.int32, (chunk, chunk), 1)
    tri = (ci <= ri).astype(jnp.float32)

    loc = jax.lax.broadcasted_iota(jnp.int32, (chunk, 1), 0)

    def body(c, base):
        sl = pl.ds(c * chunk, chunk)
        oh = oh_ref[sl, :]
        prio = _dot(tri, oh, ((1,), (0,))) + base
        myp = jnp.sum(prio * oh, axis=-1, keepdims=True)
        kslot = myp.astype(jnp.int32) - 1
        eix = eidx_ref[sl, :]
        dest = eix * cap + kslot
        routed = kslot < cap
        trash = nslot + (c * chunk + loc) // sc_chunk
        destc_ref[sl, :] = jnp.where(routed, dest, trash)
        gdest_ref[sl, :] = jnp.where(routed, dest, 0)
        routed_ref[sl, :] = routed.astype(jnp.int32)
        return base + jnp.sum(oh, axis=0, keepdims=True)

    jax.lax.fori_loop(0, s_len // chunk, body,
                      jnp.zeros((1, n_exp), jnp.float32))


def _gelu(x):
    return 0.5 * x * (1.0 + jax.lax.erf(x * 0.7071067811865476))


def _ffn_kernel(xg_ref, wi_ref, wo_ref, h_ref):
    f = pl.program_id(1)
    hmid = _gelu(_dot(xg_ref[...], wi_ref[0], ((1,), (1,))))
    part = _dot(hmid, wo_ref[0], ((1,), (1,)))

    @pl.when(f == 0)
    def _():
        h_ref[...] = part

    @pl.when(f > 0)
    def _():
        h_ref[...] = h_ref[...] + part


def _sc_disp_body(x1_hbm, destc_hbm, xg_hbm, idx_v, rows_v, sem, *,
                  sc_chunk):
    w = lax.axis_index("s") * 2 + lax.axis_index("c")
    base = w * sc_chunk
    pltpu.sync_copy(destc_hbm.at[pl.ds(base, sc_chunk)], idx_v)
    pltpu.sync_copy(x1_hbm.at[pl.ds(base, sc_chunk)], rows_v)
    pltpu.async_copy(rows_v, xg_hbm.at[idx_v], sem).wait()


def _sc_gather_body(h_hbm, gdest_hbm, hs_hbm, idx_v, rows_v, sem, *,
                    sc_chunk):
    w = lax.axis_index("s") * 2 + lax.axis_index("c")
    base = w * sc_chunk
    pltpu.sync_copy(gdest_hbm.at[pl.ds(base, sc_chunk)], idx_v)
    pltpu.async_copy(h_hbm.at[idx_v], rows_v, sem).wait()
    pltpu.sync_copy(rows_v, hs_hbm.at[pl.ds(base, sc_chunk)])


def _combine_kernel(x1_ref, maxp_ref, routed_ref, hs_ref,
                    g_ref, b_ref, o_ref, *, eps):
    routed = routed_ref[...] > 0
    nxt = jnp.where(routed, hs_ref[...], x1_ref[...])
    t = x1_ref[...] + maxp_ref[...] * nxt
    m = jnp.mean(t, axis=-1, keepdims=True)
    v = jnp.mean((t - m) * (t - m), axis=-1, keepdims=True)
    o_ref[...] = (t - m) / jnp.sqrt(v + eps) * g_ref[...] + b_ref[...]


def kernel(x, in_proj_w, in_proj_b, out_proj_w, out_proj_b,
           ln1_g, ln1_b, ln2_g, ln2_b, router_w, wi, wo):
    B, S, D = x.shape
    E = router_w.shape[0]
    FF = wi.shape[1]
    H = _H
    DH = D // H
    CAP = _CAP
    NSLOT = E * CAP
    SB = min(256, S)
    NB = S // SB
    NFF = 2
    FFB = FF // NFF

    x2 = x.reshape(S, D)

    qkv = pl.pallas_call(
        _qkv_kernel,
        grid=(NB,),
        in_specs=[
            pl.BlockSpec((SB, D), lambda i: (i, 0)),
            pl.BlockSpec((3 * D, D), lambda i: (0, 0)),
            pl.BlockSpec((1, 3 * D), lambda i: (0, 0)),
        ],
        out_specs=pl.BlockSpec((SB, 3 * D), lambda i: (i, 0)),
        out_shape=jax.ShapeDtypeStruct((S, 3 * D), jnp.float32),
        interpret=_INTERPRET,
    )(x2, in_proj_w, in_proj_b.reshape(1, 3 * D))

    HB = 128
    NHB = D // HB
    o2 = pl.pallas_call(
        functools.partial(_attn_kernel, scale=1.0 / (DH ** 0.5), dh=DH),
        grid=(NHB,),
        in_specs=[
            pl.BlockSpec((S, HB), lambda p: (0, p)),
            pl.BlockSpec((S, HB), lambda p: (0, NHB + p)),
            pl.BlockSpec((S, HB), lambda p: (0, 2 * NHB + p)),
        ],
        out_specs=pl.BlockSpec((S, HB), lambda p: (0, p)),
        out_shape=jax.ShapeDtypeStruct((S, D), jnp.float32),
        interpret=_INTERPRET,
    )(qkv, qkv, qkv)

    x1, logits = pl.pallas_call(
        functools.partial(_post_kernel, eps=_EPS),
        grid=(NB,),
        in_specs=[
            pl.BlockSpec((SB, D), lambda i: (i, 0)),
            pl.BlockSpec((SB, D), lambda i: (i, 0)),
            pl.BlockSpec((D, D), lambda i: (0, 0)),
            pl.BlockSpec((1, D), lambda i: (0, 0)),
            pl.BlockSpec((1, D), lambda i: (0, 0)),
            pl.BlockSpec((1, D), lambda i: (0, 0)),
            pl.BlockSpec((E, D), lambda i: (0, 0)),
        ],
        out_specs=[
            pl.BlockSpec((SB, D), lambda i: (i, 0)),
            pl.BlockSpec((SB, E), lambda i: (i, 0)),
        ],
        out_shape=[
            jax.ShapeDtypeStruct((S, D), jnp.float32),
            jax.ShapeDtypeStruct((S, E), jnp.float32),
        ],
        interpret=_INTERPRET,
    )(o2, x2, out_proj_w, out_proj_b.reshape(1, D), ln1_g.reshape(1, D),
      ln1_b.reshape(1, D), router_w)

    SC_CHUNK = S // 32
    XG_ROWS = (E + 1) * CAP
    eidx_c, maxp_c, destc_c, gdest_c, routed_c = pl.pallas_call(
        functools.partial(_route_kernel, cap=CAP, n_exp=E, s_len=S,
                          chunk=SB, nslot=NSLOT, sc_chunk=SC_CHUNK),
        out_shape=[
            jax.ShapeDtypeStruct((S, 1), jnp.int32),
            jax.ShapeDtypeStruct((S, 1), jnp.float32),
            jax.ShapeDtypeStruct((S, 1), jnp.int32),
            jax.ShapeDtypeStruct((S, 1), jnp.int32),
            jax.ShapeDtypeStruct((S, 1), jnp.int32),
        ],
        scratch_shapes=[pltpu.VMEM((S, E), jnp.float32)],
        interpret=_INTERPRET,
    )(logits)

    sc_mesh = plsc.VectorSubcoreMesh(core_axis_name="c",
                                     subcore_axis_name="s")
    xg = pl.kernel(
        functools.partial(_sc_disp_body, sc_chunk=SC_CHUNK),
        out_type=jax.ShapeDtypeStruct((XG_ROWS, D), jnp.float32),
        mesh=sc_mesh,
        scratch_types=[
            pltpu.VMEM((SC_CHUNK,), jnp.int32),
            pltpu.VMEM((SC_CHUNK, D), jnp.float32),
            pltpu.SemaphoreType.DMA,
        ],
    )(x1, destc_c.reshape(S))

    h = pl.pallas_call(
        _ffn_kernel,
        grid=(E, NFF),
        in_specs=[
            pl.BlockSpec((CAP, D), lambda e, f: (e, 0)),
            pl.BlockSpec((1, FFB, D), lambda e, f: (e, f, 0)),
            pl.BlockSpec((1, D, FFB), lambda e, f: (e, 0, f)),
        ],
        out_specs=pl.BlockSpec((CAP, D), lambda e, f: (e, 0)),
        out_shape=jax.ShapeDtypeStruct((NSLOT, D), jnp.float32),
        interpret=_INTERPRET,
    )(xg, wi, wo)

    hs = pl.kernel(
        functools.partial(_sc_gather_body, sc_chunk=SC_CHUNK),
        out_type=jax.ShapeDtypeStruct((S, D), jnp.float32),
        mesh=sc_mesh,
        scratch_types=[
            pltpu.VMEM((SC_CHUNK,), jnp.int32),
            pltpu.VMEM((SC_CHUNK, D), jnp.float32),
            pltpu.SemaphoreType.DMA,
        ],
    )(h, gdest_c.reshape(S))

    out = pl.pallas_call(
        functools.partial(_combine_kernel, eps=_EPS),
        grid=(NB,),
        in_specs=[
            pl.BlockSpec((SB, D), lambda i: (i, 0)),
            pl.BlockSpec((SB, 1), lambda i: (i, 0)),
            pl.BlockSpec((SB, 1), lambda i: (i, 0)),
            pl.BlockSpec((SB, D), lambda i: (i, 0)),
            pl.BlockSpec((1, D), lambda i: (0, 0)),
            pl.BlockSpec((1, D), lambda i: (0, 0)),
        ],
        out_specs=pl.BlockSpec((SB, D), lambda i: (i, 0)),
        out_shape=jax.ShapeDtypeStruct((S, D), jnp.float32),
        interpret=_INTERPRET,
    )(x1, maxp_c, routed_c, hs, ln2_g.reshape(1, D),
      ln2_b.reshape(1, D))

    return (out.reshape(B, S, D), logits.reshape(B, S, E),
            eidx_c.reshape(B, S))
